# Initial kernel scaffold; baseline (speedup 1.0000x reference)
import functools

import jax
import jax.numpy as jnp
from jax import lax
from jax.experimental import pallas as pl
from jax.experimental.pallas import tpu as pltpu
from jax.experimental.pallas import tpu_sc as plsc

N_NODES = 10000
N_EDGES = 160000
D = 128
LANES = 16

NC = 2
NS = 16
NW = NC * NS
CHUNK = 128
N_CHUNKS = N_EDGES // CHUNK
MAX_T = (N_CHUNKS + NW - 1) // NW
ROWS_PER_SUB = N_NODES // NS
ZCHUNK = 125

_MESH = plsc.VectorSubcoreMesh(core_axis_name="c", subcore_axis_name="s")


def _zero_accumulator(rows_v, acc_sh, s):
    def zrow(i, carry):
        for j in range(D // LANES):
            rows_v[i, pl.ds(j * LANES, LANES)] = jnp.zeros((LANES,), jnp.float32)
        return carry

    lax.fori_loop(0, ZCHUNK, zrow, 0)
    for z in range(ROWS_PER_SUB // ZCHUNK):
        pltpu.sync_copy(
            rows_v.at[pl.ds(0, ZCHUNK)],
            acc_sh.at[pl.ds(s * ROWS_PER_SUB + z * ZCHUNK, ZCHUNK)],
        )


def _seg_sum_h_body(prev_h, seg, out, rows_v, idx_v, acc_sh):
    c = lax.axis_index("c")
    s = lax.axis_index("s")
    w = c * NS + s

    _zero_accumulator(rows_v, acc_sh, s)
    plsc.subcore_barrier()

    def body(t, carry):
        k = w + NW * t

        @pl.when(k < N_CHUNKS)
        def _():
            base = k * CHUNK
            pltpu.sync_copy(prev_h.at[pl.ds(base, CHUNK)], rows_v)
            pltpu.sync_copy(seg.at[pl.ds(base, CHUNK)], idx_v)
            pltpu.sync_copy(rows_v, acc_sh.at[idx_v], add=True)

        return carry

    lax.fori_loop(0, MAX_T, body, 0)
    plsc.subcore_barrier()
    pltpu.sync_copy(
        acc_sh.at[pl.ds(s * ROWS_PER_SUB, ROWS_PER_SUB)],
        out.at[c, pl.ds(s * ROWS_PER_SUB, ROWS_PER_SUB)],
    )


_seg_sum_h = functools.partial(
    pl.kernel,
    out_type=jax.ShapeDtypeStruct((NC, N_NODES, D), jnp.float32),
    mesh=_MESH,
    scratch_types=[
        pltpu.VMEM((CHUNK, D), jnp.float32),
        pltpu.VMEM((CHUNK,), jnp.int32),
        pltpu.VMEM_SHARED((N_NODES, D), jnp.float32),
    ],
)(_seg_sum_h_body)


def _fc_body(fh, pc, seg, fx, out, fh_v, pc_v, fxg_v, idx_v, acc_sh, sem):
    c = lax.axis_index("c")
    s = lax.axis_index("s")
    w = c * NS + s

    _zero_accumulator(fh_v, acc_sh, s)
    plsc.subcore_barrier()

    def body(t, carry):
        k = w + NW * t

        @pl.when(k < N_CHUNKS)
        def _():
            base = k * CHUNK
            pltpu.sync_copy(fh.at[pl.ds(base, CHUNK)], fh_v)
            pltpu.sync_copy(pc.at[pl.ds(base, CHUNK)], pc_v)
            pltpu.sync_copy(seg.at[pl.ds(base, CHUNK)], idx_v)
            pltpu.async_copy(fx.at[idx_v], fxg_v, sem).wait()

            def row(i, carry2):
                for j in range(D // LANES):
                    sl = pl.ds(j * LANES, LANES)
                    x = fh_v[i, sl] + fxg_v[i, sl]
                    sg = 1.0 / (1.0 + jnp.exp(-x))
                    fh_v[i, sl] = sg * pc_v[i, sl]
                return carry2

            lax.fori_loop(0, CHUNK, row, 0)
            pltpu.sync_copy(fh_v, acc_sh.at[idx_v], add=True)

        return carry

    lax.fori_loop(0, MAX_T, body, 0)
    plsc.subcore_barrier()
    pltpu.sync_copy(
        acc_sh.at[pl.ds(s * ROWS_PER_SUB, ROWS_PER_SUB)],
        out.at[c, pl.ds(s * ROWS_PER_SUB, ROWS_PER_SUB)],
    )


_fc_partials = functools.partial(
    pl.kernel,
    out_type=jax.ShapeDtypeStruct((NC, N_NODES, D), jnp.float32),
    mesh=_MESH,
    scratch_types=[
        pltpu.VMEM((CHUNK, D), jnp.float32),
        pltpu.VMEM((CHUNK, D), jnp.float32),
        pltpu.VMEM((CHUNK, D), jnp.float32),
        pltpu.VMEM((CHUNK,), jnp.int32),
        pltpu.VMEM_SHARED((N_NODES, D), jnp.float32),
        pltpu.SemaphoreType.DMA,
    ],
)(_fc_body)


def _matmul_bias_kernel(x_ref, w_ref, b_ref, o_ref):
    o_ref[...] = (
        jnp.dot(x_ref[...], w_ref[...], preferred_element_type=jnp.float32)
        + b_ref[...]
    )


def _matmul_kernel(x_ref, w_ref, o_ref):
    o_ref[...] = jnp.dot(x_ref[...], w_ref[...], preferred_element_type=jnp.float32)


def _final_kernel(x_ref, htp_ref, fcp_ref, wc_ref, bc_ref, c_ref, h_ref):
    ht = htp_ref[0] + htp_ref[1]
    big_in = jnp.concatenate([x_ref[...], ht], axis=1)
    big = (
        jnp.dot(big_in, wc_ref[...], preferred_element_type=jnp.float32)
        + bc_ref[...]
    )
    z_i = big[:, :D]
    z_o = big[:, D : 2 * D]
    z_u = big[:, 2 * D :]
    fc = fcp_ref[0] + fcp_ref[1]
    cc = jax.nn.sigmoid(z_i) * jnp.tanh(z_u) + fc
    c_ref[...] = cc
    h_ref[...] = jax.nn.sigmoid(z_o) * jnp.tanh(cc)


@jax.jit
def kernel(inputs_mat, prev_c_mat, prev_h_mat, segment_ids, W_combined,
           b_combined, W_f, U_f, b_f):
    seg = segment_ids.astype(jnp.int32)

    f_x = pl.pallas_call(
        _matmul_bias_kernel,
        grid=(10,),
        in_specs=[
            pl.BlockSpec((1000, D), lambda i: (i, 0)),
            pl.BlockSpec((D, D), lambda i: (0, 0)),
            pl.BlockSpec((1, D), lambda i: (0, 0)),
        ],
        out_specs=pl.BlockSpec((1000, D), lambda i: (i, 0)),
        out_shape=jax.ShapeDtypeStruct((N_NODES, D), jnp.float32),
    )(inputs_mat, W_f, b_f)

    f_h = pl.pallas_call(
        _matmul_kernel,
        grid=(100,),
        in_specs=[
            pl.BlockSpec((1600, D), lambda i: (i, 0)),
            pl.BlockSpec((D, D), lambda i: (0, 0)),
        ],
        out_specs=pl.BlockSpec((1600, D), lambda i: (i, 0)),
        out_shape=jax.ShapeDtypeStruct((N_EDGES, D), jnp.float32),
    )(prev_h_mat, U_f)

    htp = _seg_sum_h(prev_h_mat, seg)

    fcp = _fc_partials(f_h, prev_c_mat, seg, f_x)

    c, h = pl.pallas_call(
        _final_kernel,
        grid=(10,),
        in_specs=[
            pl.BlockSpec((1000, D), lambda i: (i, 0)),
            pl.BlockSpec((NC, 1000, D), lambda i: (0, i, 0)),
            pl.BlockSpec((NC, 1000, D), lambda i: (0, i, 0)),
            pl.BlockSpec((2 * D, 3 * D), lambda i: (0, 0)),
            pl.BlockSpec((1, 3 * D), lambda i: (0, 0)),
        ],
        out_specs=[
            pl.BlockSpec((1000, D), lambda i: (i, 0)),
            pl.BlockSpec((1000, D), lambda i: (i, 0)),
        ],
        out_shape=[
            jax.ShapeDtypeStruct((N_NODES, D), jnp.float32),
            jax.ShapeDtypeStruct((N_NODES, D), jnp.float32),
        ],
    )(inputs_mat, htp, fcp, W_combined, b_combined)

    return (c, h)


# speedup vs baseline: 1.5230x; 1.5230x over previous
import functools

import jax
import jax.numpy as jnp
from jax import lax
from jax.experimental import pallas as pl
from jax.experimental.pallas import tpu as pltpu
from jax.experimental.pallas import tpu_sc as plsc

N_NODES = 10000
N_EDGES = 160000
D = 128
LANES = 16

NC = 2
NS = 16
NW = NC * NS
CHUNK = 128
N_CHUNKS = N_EDGES // CHUNK
MAX_T = (N_CHUNKS + NW - 1) // NW
N_PAD = 10240
ROWS_PER_SUB = N_PAD // NS

_MESH = plsc.VectorSubcoreMesh(core_axis_name="c", subcore_axis_name="s")


def _zero_accumulator(rows_v, acc_sh, s):
    def zrow(i, carry):
        for j in range(D // LANES):
            rows_v[i, pl.ds(j * LANES, LANES)] = jnp.zeros((LANES,), jnp.float32)
        return carry

    lax.fori_loop(0, CHUNK, zrow, 0)
    for z in range(ROWS_PER_SUB // CHUNK):
        pltpu.sync_copy(
            rows_v,
            acc_sh.at[pl.ds(s * ROWS_PER_SUB + z * CHUNK, CHUNK)],
        )


def _seg_sum_h_body(prev_h, seg, out, rows_v, idx_v, acc_sh):
    c = lax.axis_index("c")
    s = lax.axis_index("s")
    w = c * NS + s

    _zero_accumulator(rows_v, acc_sh, s)
    plsc.subcore_barrier()

    def body(t, carry):
        k = w + NW * t

        @pl.when(k < N_CHUNKS)
        def _():
            base = k * CHUNK
            pltpu.sync_copy(prev_h.at[pl.ds(base, CHUNK)], rows_v)
            pltpu.sync_copy(seg.at[pl.ds(base, CHUNK)], idx_v)
            pltpu.sync_copy(rows_v, acc_sh.at[idx_v], add=True)

        return carry

    lax.fori_loop(0, MAX_T, body, 0)
    plsc.subcore_barrier()
    pltpu.sync_copy(
        acc_sh.at[pl.ds(s * ROWS_PER_SUB, ROWS_PER_SUB)],
        out.at[c, pl.ds(s * ROWS_PER_SUB, ROWS_PER_SUB)],
    )


_seg_sum_h = functools.partial(
    pl.kernel,
    out_type=jax.ShapeDtypeStruct((NC, N_PAD, D), jnp.float32),
    mesh=_MESH,
    scratch_types=[
        pltpu.VMEM((CHUNK, D), jnp.float32),
        pltpu.VMEM((CHUNK,), jnp.int32),
        pltpu.VMEM_SHARED((N_PAD, D), jnp.float32),
    ],
)(_seg_sum_h_body)


def _fc_body(fh, pc, seg, fx, out, fh_v, fxg_v, idx_v, acc_sh, sem):
    c = lax.axis_index("c")
    s = lax.axis_index("s")
    w = c * NS + s

    _zero_accumulator(fh_v, acc_sh, s)
    plsc.subcore_barrier()

    def body(t, carry):
        k = w + NW * t

        @pl.when(k < N_CHUNKS)
        def _():
            base = k * CHUNK
            pltpu.sync_copy(fh.at[pl.ds(base, CHUNK)], fh_v)
            pltpu.sync_copy(seg.at[pl.ds(base, CHUNK)], idx_v)
            pltpu.async_copy(fx.at[idx_v], fxg_v, sem).wait()

            def row(i, carry2):
                for j in range(D // LANES):
                    sl = pl.ds(j * LANES, LANES)
                    x = fh_v[i, sl] + fxg_v[i, sl]
                    fh_v[i, sl] = 1.0 / (1.0 + jnp.exp(-x))
                return carry2

            lax.fori_loop(0, CHUNK, row, 0)
            pltpu.sync_copy(pc.at[pl.ds(base, CHUNK)], fxg_v)

            def row2(i, carry2):
                for j in range(D // LANES):
                    sl = pl.ds(j * LANES, LANES)
                    fh_v[i, sl] = fh_v[i, sl] * fxg_v[i, sl]
                return carry2

            lax.fori_loop(0, CHUNK, row2, 0)
            pltpu.sync_copy(fh_v, acc_sh.at[idx_v], add=True)

        return carry

    lax.fori_loop(0, MAX_T, body, 0)
    plsc.subcore_barrier()
    pltpu.sync_copy(
        acc_sh.at[pl.ds(s * ROWS_PER_SUB, ROWS_PER_SUB)],
        out.at[c, pl.ds(s * ROWS_PER_SUB, ROWS_PER_SUB)],
    )


_fc_partials = functools.partial(
    pl.kernel,
    out_type=jax.ShapeDtypeStruct((NC, N_PAD, D), jnp.float32),
    mesh=_MESH,
    scratch_types=[
        pltpu.VMEM((CHUNK, D), jnp.float32),
        pltpu.VMEM((CHUNK, D), jnp.float32),
        pltpu.VMEM((CHUNK,), jnp.int32),
        pltpu.VMEM_SHARED((N_PAD, D), jnp.float32),
        pltpu.SemaphoreType.DMA,
    ],
)(_fc_body)


def _matmul_bias_kernel(x_ref, w_ref, b_ref, o_ref):
    o_ref[...] = (
        jnp.dot(x_ref[...], w_ref[...], preferred_element_type=jnp.float32)
        + b_ref[...]
    )


def _matmul_kernel(x_ref, w_ref, o_ref):
    o_ref[...] = jnp.dot(x_ref[...], w_ref[...], preferred_element_type=jnp.float32)


def _final_kernel(x_ref, htp_ref, fcp_ref, wc_ref, bc_ref, c_ref, h_ref):
    ht = htp_ref[0] + htp_ref[1]
    big_in = jnp.concatenate([x_ref[...], ht], axis=1)
    big = (
        jnp.dot(big_in, wc_ref[...], preferred_element_type=jnp.float32)
        + bc_ref[...]
    )
    z_i = big[:, :D]
    z_o = big[:, D : 2 * D]
    z_u = big[:, 2 * D :]
    fc = fcp_ref[0] + fcp_ref[1]
    cc = jax.nn.sigmoid(z_i) * jnp.tanh(z_u) + fc
    c_ref[...] = cc
    h_ref[...] = jax.nn.sigmoid(z_o) * jnp.tanh(cc)


@jax.jit
def kernel(inputs_mat, prev_c_mat, prev_h_mat, segment_ids, W_combined,
           b_combined, W_f, U_f, b_f):
    seg = segment_ids.astype(jnp.int32)

    f_x = pl.pallas_call(
        _matmul_bias_kernel,
        grid=(10,),
        in_specs=[
            pl.BlockSpec((1000, D), lambda i: (i, 0)),
            pl.BlockSpec((D, D), lambda i: (0, 0)),
            pl.BlockSpec((1, D), lambda i: (0, 0)),
        ],
        out_specs=pl.BlockSpec((1000, D), lambda i: (i, 0)),
        out_shape=jax.ShapeDtypeStruct((N_NODES, D), jnp.float32),
    )(inputs_mat, W_f, b_f)

    f_h = pl.pallas_call(
        _matmul_kernel,
        grid=(100,),
        in_specs=[
            pl.BlockSpec((1600, D), lambda i: (i, 0)),
            pl.BlockSpec((D, D), lambda i: (0, 0)),
        ],
        out_specs=pl.BlockSpec((1600, D), lambda i: (i, 0)),
        out_shape=jax.ShapeDtypeStruct((N_EDGES, D), jnp.float32),
    )(prev_h_mat, U_f)

    htp = _seg_sum_h(prev_h_mat, seg)

    fcp = _fc_partials(f_h, prev_c_mat, seg, f_x)

    c, h = pl.pallas_call(
        _final_kernel,
        grid=(10,),
        in_specs=[
            pl.BlockSpec((1000, D), lambda i: (i, 0)),
            pl.BlockSpec((NC, 1000, D), lambda i: (0, i, 0)),
            pl.BlockSpec((NC, 1000, D), lambda i: (0, i, 0)),
            pl.BlockSpec((2 * D, 3 * D), lambda i: (0, 0)),
            pl.BlockSpec((1, 3 * D), lambda i: (0, 0)),
        ],
        out_specs=[
            pl.BlockSpec((1000, D), lambda i: (i, 0)),
            pl.BlockSpec((1000, D), lambda i: (i, 0)),
        ],
        out_shape=[
            jax.ShapeDtypeStruct((N_NODES, D), jnp.float32),
            jax.ShapeDtypeStruct((N_NODES, D), jnp.float32),
        ],
    )(inputs_mat, htp, fcp, W_combined, b_combined)

    return (c, h)


# speedup vs baseline: 1.9658x; 1.2907x over previous
import functools

import jax
import jax.numpy as jnp
from jax import lax
from jax.experimental import pallas as pl
from jax.experimental.pallas import tpu as pltpu
from jax.experimental.pallas import tpu_sc as plsc

N_NODES = 10000
N_EDGES = 160000
D = 128
LANES = 16

NC = 2
NS = 16
NW = NC * NS
CHUNK = 128
N_CHUNKS = N_EDGES // CHUNK
MAX_T = (N_CHUNKS + NW - 1) // NW
N_PAD = 10112
ROWS_PER_SUB = N_PAD // NS

_MESH = plsc.VectorSubcoreMesh(core_axis_name="c", subcore_axis_name="s")


def _zero_accumulator(rows_v, acc_sh, s):
    def zrow(i, carry):
        for j in range(D // LANES):
            rows_v[i, pl.ds(j * LANES, LANES)] = jnp.zeros((LANES,), jnp.float32)
        return carry

    lax.fori_loop(0, CHUNK, zrow, 0)
    nfull, rem = divmod(ROWS_PER_SUB, CHUNK)
    for z in range(nfull):
        pltpu.sync_copy(
            rows_v,
            acc_sh.at[pl.ds(s * ROWS_PER_SUB + z * CHUNK, CHUNK)],
        )
    if rem:
        pltpu.sync_copy(
            rows_v.at[pl.ds(0, rem)],
            acc_sh.at[pl.ds(s * ROWS_PER_SUB + nfull * CHUNK, rem)],
        )


def _seg_sum_h_body(prev_h, seg, out, rows_v, idx_v, acc_sh, sem):
    c = lax.axis_index("c")
    s = lax.axis_index("s")
    w = c * NS + s

    _zero_accumulator(rows_v, acc_sh, s)
    plsc.subcore_barrier()

    def body(t, carry):
        k = w + NW * t

        @pl.when(k < N_CHUNKS)
        def _():
            base = k * CHUNK
            cp = pltpu.async_copy(prev_h.at[pl.ds(base, CHUNK)], rows_v, sem)
            pltpu.sync_copy(seg.at[pl.ds(base, CHUNK)], idx_v)
            cp.wait()
            pltpu.sync_copy(rows_v, acc_sh.at[idx_v], add=True)

        return carry

    lax.fori_loop(0, MAX_T, body, 0)
    plsc.subcore_barrier()
    pltpu.sync_copy(
        acc_sh.at[pl.ds(s * ROWS_PER_SUB, ROWS_PER_SUB)],
        out.at[c, pl.ds(s * ROWS_PER_SUB, ROWS_PER_SUB)],
    )


_seg_sum_h = functools.partial(
    pl.kernel,
    out_type=jax.ShapeDtypeStruct((NC, N_PAD, D), jnp.float32),
    mesh=_MESH,
    scratch_types=[
        pltpu.VMEM((CHUNK, D), jnp.float32),
        pltpu.VMEM((CHUNK,), jnp.int32),
        pltpu.VMEM_SHARED((N_PAD, D), jnp.float32),
        pltpu.SemaphoreType.DMA,
    ],
)(_seg_sum_h_body)


def _fc_body(fh, pc, seg, fx, out, fh_v, pc_v, fxg_v, idx_v, acc_sh, sem, sem2, sem3):
    c = lax.axis_index("c")
    s = lax.axis_index("s")
    w = c * NS + s

    _zero_accumulator(fh_v, acc_sh, s)
    plsc.subcore_barrier()

    def body(t, carry):
        k = w + NW * t

        @pl.when(k < N_CHUNKS)
        def _():
            base = k * CHUNK
            cp1 = pltpu.async_copy(fh.at[pl.ds(base, CHUNK)], fh_v, sem)
            cp2 = pltpu.async_copy(pc.at[pl.ds(base, CHUNK)], pc_v, sem2)
            pltpu.sync_copy(seg.at[pl.ds(base, CHUNK)], idx_v)
            cp3 = pltpu.async_copy(fx.at[idx_v], fxg_v, sem3)
            cp1.wait()
            cp2.wait()
            cp3.wait()

            def row(i, carry2):
                for j in range(D // LANES):
                    sl = pl.ds(j * LANES, LANES)
                    x = fh_v[i, sl] + fxg_v[i, sl]
                    sg = 1.0 / (1.0 + jnp.exp(-x))
                    fh_v[i, sl] = sg * pc_v[i, sl]
                return carry2

            lax.fori_loop(0, CHUNK, row, 0)
            pltpu.sync_copy(fh_v, acc_sh.at[idx_v], add=True)

        return carry

    lax.fori_loop(0, MAX_T, body, 0)
    plsc.subcore_barrier()
    pltpu.sync_copy(
        acc_sh.at[pl.ds(s * ROWS_PER_SUB, ROWS_PER_SUB)],
        out.at[c, pl.ds(s * ROWS_PER_SUB, ROWS_PER_SUB)],
    )


_fc_partials = functools.partial(
    pl.kernel,
    out_type=jax.ShapeDtypeStruct((NC, N_PAD, D), jnp.float32),
    mesh=_MESH,
    scratch_types=[
        pltpu.VMEM((CHUNK, D), jnp.float32),
        pltpu.VMEM((CHUNK, D), jnp.float32),
        pltpu.VMEM((CHUNK, D), jnp.float32),
        pltpu.VMEM((CHUNK,), jnp.int32),
        pltpu.VMEM_SHARED((N_PAD, D), jnp.float32),
        pltpu.SemaphoreType.DMA,
        pltpu.SemaphoreType.DMA,
        pltpu.SemaphoreType.DMA,
    ],
)(_fc_body)


def _matmul_bias_kernel(x_ref, w_ref, b_ref, o_ref):
    o_ref[...] = (
        jnp.dot(x_ref[...], w_ref[...], preferred_element_type=jnp.float32)
        + b_ref[...]
    )


def _matmul_kernel(x_ref, w_ref, o_ref):
    o_ref[...] = jnp.dot(x_ref[...], w_ref[...], preferred_element_type=jnp.float32)


def _final_kernel(x_ref, htp_ref, fcp_ref, wc_ref, bc_ref, c_ref, h_ref):
    ht = htp_ref[0] + htp_ref[1]
    big_in = jnp.concatenate([x_ref[...], ht], axis=1)
    big = (
        jnp.dot(big_in, wc_ref[...], preferred_element_type=jnp.float32)
        + bc_ref[...]
    )
    z_i = big[:, :D]
    z_o = big[:, D : 2 * D]
    z_u = big[:, 2 * D :]
    fc = fcp_ref[0] + fcp_ref[1]
    cc = jax.nn.sigmoid(z_i) * jnp.tanh(z_u) + fc
    c_ref[...] = cc
    h_ref[...] = jax.nn.sigmoid(z_o) * jnp.tanh(cc)


@jax.jit
def kernel(inputs_mat, prev_c_mat, prev_h_mat, segment_ids, W_combined,
           b_combined, W_f, U_f, b_f):
    seg = segment_ids.astype(jnp.int32)

    htp = _seg_sum_h(prev_h_mat, seg)

    f_x = pl.pallas_call(
        _matmul_bias_kernel,
        grid=(10,),
        in_specs=[
            pl.BlockSpec((1000, D), lambda i: (i, 0)),
            pl.BlockSpec((D, D), lambda i: (0, 0)),
            pl.BlockSpec((1, D), lambda i: (0, 0)),
        ],
        out_specs=pl.BlockSpec((1000, D), lambda i: (i, 0)),
        out_shape=jax.ShapeDtypeStruct((N_NODES, D), jnp.float32),
    )(inputs_mat, W_f, b_f)

    f_h = pl.pallas_call(
        _matmul_kernel,
        grid=(100,),
        in_specs=[
            pl.BlockSpec((1600, D), lambda i: (i, 0)),
            pl.BlockSpec((D, D), lambda i: (0, 0)),
        ],
        out_specs=pl.BlockSpec((1600, D), lambda i: (i, 0)),
        out_shape=jax.ShapeDtypeStruct((N_EDGES, D), jnp.float32),
    )(prev_h_mat, U_f)

    fcp = _fc_partials(f_h, prev_c_mat, seg, f_x)

    c, h = pl.pallas_call(
        _final_kernel,
        grid=(10,),
        in_specs=[
            pl.BlockSpec((1000, D), lambda i: (i, 0)),
            pl.BlockSpec((NC, 1000, D), lambda i: (0, i, 0)),
            pl.BlockSpec((NC, 1000, D), lambda i: (0, i, 0)),
            pl.BlockSpec((2 * D, 3 * D), lambda i: (0, 0)),
            pl.BlockSpec((1, 3 * D), lambda i: (0, 0)),
        ],
        out_specs=[
            pl.BlockSpec((1000, D), lambda i: (i, 0)),
            pl.BlockSpec((1000, D), lambda i: (i, 0)),
        ],
        out_shape=[
            jax.ShapeDtypeStruct((N_NODES, D), jnp.float32),
            jax.ShapeDtypeStruct((N_NODES, D), jnp.float32),
        ],
    )(inputs_mat, htp, fcp, W_combined, b_combined)

    return (c, h)


# speedup vs baseline: 2.0917x; 1.0641x over previous
import functools

import jax
import jax.numpy as jnp
from jax import lax
from jax.experimental import pallas as pl
from jax.experimental.pallas import tpu as pltpu
from jax.experimental.pallas import tpu_sc as plsc

N_NODES = 10000
N_EDGES = 160000
D = 128
LANES = 16

NC = 2
NS = 16
NW = NC * NS
CHUNK = 128
N_CHUNKS = N_EDGES // CHUNK
MAX_T = (N_CHUNKS + NW - 1) // NW
N_PAD = 10112
ROWS_PER_SUB = N_PAD // NS

_MESH = plsc.VectorSubcoreMesh(core_axis_name="c", subcore_axis_name="s")


def _zero_accumulator(rows_v, acc_sh, s):
    def zrow(i, carry):
        for j in range(D // LANES):
            rows_v[i, pl.ds(j * LANES, LANES)] = jnp.zeros((LANES,), jnp.float32)
        return carry

    lax.fori_loop(0, CHUNK, zrow, 0)
    nfull, rem = divmod(ROWS_PER_SUB, CHUNK)
    for z in range(nfull):
        pltpu.sync_copy(
            rows_v,
            acc_sh.at[pl.ds(s * ROWS_PER_SUB + z * CHUNK, CHUNK)],
        )
    if rem:
        pltpu.sync_copy(
            rows_v.at[pl.ds(0, rem)],
            acc_sh.at[pl.ds(s * ROWS_PER_SUB + nfull * CHUNK, rem)],
        )


def _seg_sum_h_body(prev_h, seg, out, rows_v, idx_v, acc_sh, sem):
    c = lax.axis_index("c")
    s = lax.axis_index("s")
    w = c * NS + s

    _zero_accumulator(rows_v, acc_sh, s)
    plsc.subcore_barrier()

    def body(t, carry):
        k = w + NW * t

        @pl.when(k < N_CHUNKS)
        def _():
            base = k * CHUNK
            cp = pltpu.async_copy(prev_h.at[pl.ds(base, CHUNK)], rows_v, sem)
            pltpu.sync_copy(seg.at[pl.ds(base, CHUNK)], idx_v)
            cp.wait()
            pltpu.sync_copy(rows_v, acc_sh.at[idx_v], add=True)

        return carry

    lax.fori_loop(0, MAX_T, body, 0)
    plsc.subcore_barrier()
    pltpu.sync_copy(
        acc_sh.at[pl.ds(s * ROWS_PER_SUB, ROWS_PER_SUB)],
        out.at[c, pl.ds(s * ROWS_PER_SUB, ROWS_PER_SUB)],
    )


_seg_sum_h = functools.partial(
    pl.kernel,
    out_type=jax.ShapeDtypeStruct((NC, N_PAD, D), jnp.float32),
    mesh=_MESH,
    scratch_types=[
        pltpu.VMEM((CHUNK, D), jnp.float32),
        pltpu.VMEM((CHUNK,), jnp.int32),
        pltpu.VMEM_SHARED((N_PAD, D), jnp.float32),
        pltpu.SemaphoreType.DMA,
    ],
)(_seg_sum_h_body)


def _fc_body(fh, pc, seg, fx, out, fh_v, pc_v, fxg_v, idx_v, acc_sh, sem, sem2, sem3):
    c = lax.axis_index("c")
    s = lax.axis_index("s")
    w = c * NS + s

    _zero_accumulator(fh_v, acc_sh, s)
    plsc.subcore_barrier()

    def body(t, carry):
        k = w + NW * t

        @pl.when(k < N_CHUNKS)
        def _():
            base = k * CHUNK
            cp1 = pltpu.async_copy(fh.at[pl.ds(base, CHUNK)], fh_v, sem)
            cp2 = pltpu.async_copy(pc.at[pl.ds(base, CHUNK)], pc_v, sem2)
            pltpu.sync_copy(seg.at[pl.ds(base, CHUNK)], idx_v)
            cp3 = pltpu.async_copy(fx.at[idx_v], fxg_v, sem3)
            cp1.wait()
            cp2.wait()
            cp3.wait()

            @plsc.parallel_loop(0, CHUNK, 1, unroll=2)
            def row(i):
                for j in range(D // LANES):
                    sl = pl.ds(j * LANES, LANES)
                    x = fh_v[i, sl] + fxg_v[i, sl]
                    sg = 1.0 / (1.0 + jnp.exp(-x))
                    fh_v[i, sl] = sg * pc_v[i, sl]
            pltpu.sync_copy(fh_v, acc_sh.at[idx_v], add=True)

        return carry

    lax.fori_loop(0, MAX_T, body, 0)
    plsc.subcore_barrier()
    pltpu.sync_copy(
        acc_sh.at[pl.ds(s * ROWS_PER_SUB, ROWS_PER_SUB)],
        out.at[c, pl.ds(s * ROWS_PER_SUB, ROWS_PER_SUB)],
    )


_fc_partials = functools.partial(
    pl.kernel,
    out_type=jax.ShapeDtypeStruct((NC, N_PAD, D), jnp.float32),
    mesh=_MESH,
    scratch_types=[
        pltpu.VMEM((CHUNK, D), jnp.float32),
        pltpu.VMEM((CHUNK, D), jnp.float32),
        pltpu.VMEM((CHUNK, D), jnp.float32),
        pltpu.VMEM((CHUNK,), jnp.int32),
        pltpu.VMEM_SHARED((N_PAD, D), jnp.float32),
        pltpu.SemaphoreType.DMA,
        pltpu.SemaphoreType.DMA,
        pltpu.SemaphoreType.DMA,
    ],
)(_fc_body)


def _matmul_bias_kernel(x_ref, w_ref, b_ref, o_ref):
    o_ref[...] = (
        jnp.dot(x_ref[...], w_ref[...], preferred_element_type=jnp.float32)
        + b_ref[...]
    )


def _matmul_kernel(x_ref, w_ref, o_ref):
    o_ref[...] = jnp.dot(x_ref[...], w_ref[...], preferred_element_type=jnp.float32)


def _final_kernel(x_ref, htp_ref, fcp_ref, wc_ref, bc_ref, c_ref, h_ref):
    ht = htp_ref[0] + htp_ref[1]
    big_in = jnp.concatenate([x_ref[...], ht], axis=1)
    big = (
        jnp.dot(big_in, wc_ref[...], preferred_element_type=jnp.float32)
        + bc_ref[...]
    )
    z_i = big[:, :D]
    z_o = big[:, D : 2 * D]
    z_u = big[:, 2 * D :]
    fc = fcp_ref[0] + fcp_ref[1]
    cc = jax.nn.sigmoid(z_i) * jnp.tanh(z_u) + fc
    c_ref[...] = cc
    h_ref[...] = jax.nn.sigmoid(z_o) * jnp.tanh(cc)


@jax.jit
def kernel(inputs_mat, prev_c_mat, prev_h_mat, segment_ids, W_combined,
           b_combined, W_f, U_f, b_f):
    seg = segment_ids.astype(jnp.int32)

    htp = _seg_sum_h(prev_h_mat, seg)

    f_x = pl.pallas_call(
        _matmul_bias_kernel,
        grid=(10,),
        in_specs=[
            pl.BlockSpec((1000, D), lambda i: (i, 0)),
            pl.BlockSpec((D, D), lambda i: (0, 0)),
            pl.BlockSpec((1, D), lambda i: (0, 0)),
        ],
        out_specs=pl.BlockSpec((1000, D), lambda i: (i, 0)),
        out_shape=jax.ShapeDtypeStruct((N_NODES, D), jnp.float32),
    )(inputs_mat, W_f, b_f)

    f_h = pl.pallas_call(
        _matmul_kernel,
        grid=(100,),
        in_specs=[
            pl.BlockSpec((1600, D), lambda i: (i, 0)),
            pl.BlockSpec((D, D), lambda i: (0, 0)),
        ],
        out_specs=pl.BlockSpec((1600, D), lambda i: (i, 0)),
        out_shape=jax.ShapeDtypeStruct((N_EDGES, D), jnp.float32),
    )(prev_h_mat, U_f)

    fcp = _fc_partials(f_h, prev_c_mat, seg, f_x)

    c, h = pl.pallas_call(
        _final_kernel,
        grid=(10,),
        in_specs=[
            pl.BlockSpec((1000, D), lambda i: (i, 0)),
            pl.BlockSpec((NC, 1000, D), lambda i: (0, i, 0)),
            pl.BlockSpec((NC, 1000, D), lambda i: (0, i, 0)),
            pl.BlockSpec((2 * D, 3 * D), lambda i: (0, 0)),
            pl.BlockSpec((1, 3 * D), lambda i: (0, 0)),
        ],
        out_specs=[
            pl.BlockSpec((1000, D), lambda i: (i, 0)),
            pl.BlockSpec((1000, D), lambda i: (i, 0)),
        ],
        out_shape=[
            jax.ShapeDtypeStruct((N_NODES, D), jnp.float32),
            jax.ShapeDtypeStruct((N_NODES, D), jnp.float32),
        ],
    )(inputs_mat, htp, fcp, W_combined, b_combined)

    return (c, h)
